# Initial kernel scaffold; baseline (speedup 1.0000x reference)
import jax
import jax.numpy as jnp
from jax.experimental import pallas as pl


def kernel(input_seq, input_title, edge_index, g_node_feature, edge_type, edge_norm, emb_table, conv_w, conv_b, cf_w, cf_b, rgcn_basis0, rgcn_comp0, rgcn_bias0, rgcn_basis1, rgcn_comp1, rgcn_bias1, rgcn_basis2, rgcn_comp2, rgcn_bias2, cor_w1_0, cor_b1_0, cor_w2_0, cor_b2_0, cor_w1_1, cor_b1_1, cor_w2_1, cor_b2_1):
    raise NotImplementedError("write your pallas kernel here")


# speedup vs baseline: 3.8134x; 3.8134x over previous
import functools

import jax
import jax.numpy as jnp
from jax import lax
from jax.experimental import pallas as pl
from jax.experimental.pallas import tpu as pltpu
from jax.experimental.pallas import tpu_sc as plsc

N = 10000
EMB = 200
NK = 200
D = 224
DH = 112
E = 160000
B = 4
COR = 1000
NC = 2
NS = 16
NW = NC * NS
CH = 128
NCHUNK = E // CH
SUB_ROWS = 624
ZCH = 104

_f32 = jnp.float32
_i32 = jnp.int32


def _emb_gather(table, ids):
    G = ids.shape[0]
    per = G // NW
    mesh = plsc.VectorSubcoreMesh(core_axis_name="c", subcore_axis_name="s")

    @functools.partial(
        pl.kernel,
        out_type=jax.ShapeDtypeStruct((G, EMB), _f32),
        mesh=mesh,
        scratch_types=[
            pltpu.VMEM((per,), _i32),
            pltpu.VMEM((per, EMB), _f32),
            pltpu.SemaphoreType.DMA,
        ],
        compiler_params=pltpu.CompilerParams(use_tc_tiling_on_sc=False),
    )
    def k(table_h, ids_h, out_h, idxv, rowsv, sem):
        c = lax.axis_index("c")
        s = lax.axis_index("s")
        w = c * NS + s
        base = w * per
        pltpu.sync_copy(ids_h.at[pl.ds(base, per)], idxv)
        pltpu.async_copy(table_h.at[idxv], rowsv, sem).wait()
        pltpu.sync_copy(rowsv, out_h.at[pl.ds(base, per)])

    return k(table, ids)


def _rgcn_scatter(hr0, hr1, key, dst, norm):
    mesh = plsc.VectorSubcoreMesh(core_axis_name="c", subcore_axis_name="s")

    @functools.partial(
        pl.kernel,
        out_type=jax.ShapeDtypeStruct((N, D), _f32),
        mesh=mesh,
        scratch_types=[
            pltpu.VMEM_SHARED((N, DH), _f32),
            pltpu.VMEM((CH,), _i32),
            pltpu.VMEM((CH,), _i32),
            pltpu.VMEM((CH,), _f32),
            pltpu.VMEM((CH, DH), _f32),
            pltpu.SemaphoreType.DMA,
        ],
        compiler_params=pltpu.CompilerParams(use_tc_tiling_on_sc=False),
    )
    def k(hr0_h, hr1_h, key_h, dst_h, norm_h, out_h,
          shared, keyv, dstv, normv, buf, sem):
        c = lax.axis_index("c")
        s = lax.axis_index("s")

        zero = jnp.zeros((16,), _f32)

        def zrow(r, _):
            for k2 in range(DH // 16):
                buf[r, pl.ds(k2 * 16, 16)] = zero
            return 0

        lax.fori_loop(0, ZCH, zrow, 0)
        row0 = s * SUB_ROWS
        for j in range(SUB_ROWS // ZCH):
            pltpu.sync_copy(buf.at[pl.ds(0, ZCH)],
                            shared.at[pl.ds(row0 + j * ZCH, ZCH)])

        @pl.when(s == NS - 1)
        def _():
            pltpu.sync_copy(buf.at[pl.ds(0, 16)],
                            shared.at[pl.ds(NS * SUB_ROWS, 16)])

        plsc.subcore_barrier()

        n_extra = NCHUNK - (NCHUNK // NS) * NS
        trips = jnp.where(s < n_extra, NCHUNK // NS + 1, NCHUNK // NS)

        def step(t, _):
            off = (s + t * NS) * CH
            pltpu.sync_copy(key_h.at[pl.ds(off, CH)], keyv)
            pltpu.sync_copy(dst_h.at[pl.ds(off, CH)], dstv)
            pltpu.sync_copy(norm_h.at[pl.ds(off, CH)], normv)

            @pl.when(c == 0)
            def _():
                pltpu.async_copy(hr0_h.at[keyv], buf, sem).wait()

            @pl.when(c == 1)
            def _():
                pltpu.async_copy(hr1_h.at[keyv], buf, sem).wait()

            def sgroup(g, _):
                nvv = normv[pl.ds(g * 16, 16)]
                dnums = lax.GatherDimensionNumbers(
                    offset_dims=(), collapsed_slice_dims=(0,),
                    start_index_map=(0,))
                for lane in range(16):
                    nv = lax.gather(
                        nvv, jnp.full((16, 1), lane, _i32), dnums,
                        slice_sizes=(1,),
                        mode=lax.GatherScatterMode.PROMISE_IN_BOUNDS)
                    r = g * 16 + lane
                    for k2 in range(DH // 16):
                        sl = pl.ds(k2 * 16, 16)
                        buf[r, sl] = buf[r, sl] * nv
                return 0

            lax.fori_loop(0, CH // 16, sgroup, 0)
            pltpu.sync_copy(buf, shared.at[dstv], add=True)
            return 0

        lax.fori_loop(0, trips, step, 0)
        plsc.subcore_barrier()

        col0 = c * DH
        for j in range(SUB_ROWS // ZCH):
            pltpu.sync_copy(
                shared.at[pl.ds(row0 + j * ZCH, ZCH)],
                out_h.at[pl.ds(row0 + j * ZCH, ZCH), pl.ds(col0, DH)])

        @pl.when(s == NS - 1)
        def _():
            pltpu.sync_copy(
                shared.at[pl.ds(NS * SUB_ROWS, 16)],
                out_h.at[pl.ds(NS * SUB_ROWS, 16), pl.ds(col0, DH)])

    return k(hr0, hr1, key, dst, norm)


def _conv(eseq, wk, bias):
    L = eseq.shape[1]
    LP = L - 2

    def body(e_ref, w_ref, b_ref, out_ref):
        for b in range(B):
            acc = jnp.dot(e_ref[b, 0:LP, :], w_ref[0],
                          preferred_element_type=_f32)
            acc += jnp.dot(e_ref[b, 1:LP + 1, :], w_ref[1],
                           preferred_element_type=_f32)
            acc += jnp.dot(e_ref[b, 2:LP + 2, :], w_ref[2],
                           preferred_element_type=_f32)
            acc += b_ref[...]
            out_ref[b] = jnp.maximum(acc, 0.0)

    return pl.pallas_call(
        body,
        out_shape=jax.ShapeDtypeStruct((B, LP, NK), _f32),
    )(eseq, wk, bias)


def _rgcn_mm(p, basis, comp, bias, first):
    R = 2000
    grid = (N // R,)

    def body(p_ref, basis_ref, comp_ref, bias_ref, o0_ref, o1_ref):
        if first:
            h = p_ref[...]
        else:
            h = jnp.maximum(p_ref[...] + bias_ref[...], 0.0)
        b0 = basis_ref[0]
        b1 = basis_ref[1]
        w0 = comp_ref[0, 0] * b0 + comp_ref[0, 1] * b1
        w1 = comp_ref[1, 0] * b0 + comp_ref[1, 1] * b1
        r0 = jnp.dot(h, w0, preferred_element_type=_f32)
        r1 = jnp.dot(h, w1, preferred_element_type=_f32)
        o0_ref[0] = r0[:, :DH]
        o0_ref[1] = r1[:, :DH]
        o1_ref[0] = r0[:, DH:]
        o1_ref[1] = r1[:, DH:]

    din = basis.shape[1]
    if first:
        p_spec = pl.BlockSpec((R, din), lambda i: (i, 0))
    else:
        p_spec = pl.BlockSpec((R, D), lambda i: (i, 0))
    half_spec = pl.BlockSpec((2, R, DH), lambda i: (0, i, 0))
    hr0, hr1 = pl.pallas_call(
        body,
        grid=grid,
        in_specs=[
            p_spec,
            pl.BlockSpec((2, din, D), lambda i: (0, 0, 0)),
            pl.BlockSpec((2, 2), lambda i: (0, 0)),
            pl.BlockSpec((1, D), lambda i: (0, 0)),
        ],
        out_specs=[half_spec, half_spec],
        out_shape=[jax.ShapeDtypeStruct((2, N, DH), _f32),
                   jax.ShapeDtypeStruct((2, N, DH), _f32)],
    )(p, basis, comp, bias)
    return hr0.reshape(2 * N, DH), hr1.reshape(2 * N, DH)


def _att(ac, tcv, g, p2, bias2, cf_w, cf_b):
    BN = 2000
    grid = (N // BN,)
    LA = ac.shape[1]
    LT = tcv.shape[1]

    def body(ac_ref, tc_ref, g_ref, p2_ref, b2_ref, cw_ref, cb_ref, out_ref):
        g_blk = g_ref[...]
        h3 = p2_ref[...] + b2_ref[...]
        lf = jnp.concatenate([h3[:, :EMB], g_blk], axis=1)
        cw = cw_ref[...]
        cb = cb_ref[...]
        for b in range(B):
            sa = lax.dot_general(ac_ref[b], g_blk,
                                 (((1,), (1,)), ((), ())),
                                 preferred_element_type=_f32)
            sa = jnp.exp(sa - jnp.max(sa, axis=0, keepdims=True))
            att_a = sa / jnp.sum(sa, axis=0, keepdims=True)
            ca = lax.dot_general(ac_ref[b], att_a,
                                 (((0,), (0,)), ((), ())),
                                 preferred_element_type=_f32)
            st = lax.dot_general(tc_ref[b], g_blk,
                                 (((1,), (1,)), ((), ())),
                                 preferred_element_type=_f32)
            st = jnp.exp(st - jnp.max(st, axis=0, keepdims=True))
            att_t = st / jnp.sum(st, axis=0, keepdims=True)
            ct = lax.dot_general(tc_ref[b], att_t,
                                 (((0,), (0,)), ((), ())),
                                 preferred_element_type=_f32)
            cc = jnp.concatenate([ca, ct], axis=0)
            xf = lax.dot_general(cc, cw,
                                 (((0,), (1,)), ((), ())),
                                 preferred_element_type=_f32)
            xf = jnp.tanh(xf + cb)
            out_ref[0, b, :] = jnp.sum(xf * lf, axis=1)

    return pl.pallas_call(
        body,
        grid=grid,
        in_specs=[
            pl.BlockSpec((B, LA, NK), lambda i: (0, 0, 0)),
            pl.BlockSpec((B, LT, NK), lambda i: (0, 0, 0)),
            pl.BlockSpec((BN, EMB), lambda i: (i, 0)),
            pl.BlockSpec((BN, D), lambda i: (i, 0)),
            pl.BlockSpec((1, D), lambda i: (0, 0)),
            pl.BlockSpec((2 * EMB, 2 * NK), lambda i: (0, 0)),
            pl.BlockSpec((1, 2 * EMB), lambda i: (0, 0)),
        ],
        out_specs=pl.BlockSpec((1, B, BN), lambda i: (i, 0, 0)),
        out_shape=jax.ShapeDtypeStruct((N // BN, B, BN), _f32),
    )(ac, tcv, g, p2, bias2, cf_w, cf_b)


NB5 = 5
BN5 = N // NB5


def _cor_reduce(x5, w1r, b1):
    CB = 200
    grid = (COR // CB,)

    def body(x_ref, w_ref, b_ref, out_ref):
        acc = jnp.zeros((B, CB), _f32)
        for k in range(NB5):
            o = jax.nn.sigmoid(x_ref[k])
            acc += lax.dot_general(o, w_ref[:, k, :],
                                   (((1,), (1,)), ((), ())),
                                   preferred_element_type=_f32)
        a = acc + b_ref[0, 0][None, :]
        out_ref[0] = jnp.where(a > 0, a, jnp.exp(a) - 1.0)

    return pl.pallas_call(
        body,
        grid=grid,
        in_specs=[
            pl.BlockSpec((NB5, B, BN5), lambda i: (0, 0, 0)),
            pl.BlockSpec((CB, NB5, BN5), lambda i: (i, 0, 0)),
            pl.BlockSpec((1, 1, CB), lambda i: (i, 0, 0)),
        ],
        out_specs=pl.BlockSpec((1, B, CB), lambda i: (i, 0, 0)),
        out_shape=jax.ShapeDtypeStruct((COR // CB, B, CB), _f32),
    )(x5, w1r, b1)


def _cor_expand(c, w2, b2r, x5, final):
    grid = (NB5,)

    def body(c_ref, w_ref, b_ref, x_ref, out_ref):
        r = lax.dot_general(c_ref[...], w_ref[...], (((1,), (1,)), ((), ())),
                            preferred_element_type=_f32)
        r = r + b_ref[0] + x_ref[0]
        if final:
            r = jax.nn.sigmoid(r)
        out_ref[0] = r

    return pl.pallas_call(
        body,
        grid=grid,
        in_specs=[
            pl.BlockSpec((B, COR), lambda i: (0, 0)),
            pl.BlockSpec((BN5, COR), lambda i: (i, 0)),
            pl.BlockSpec((1, 1, BN5), lambda i: (i, 0, 0)),
            pl.BlockSpec((1, B, BN5), lambda i: (i, 0, 0)),
        ],
        out_specs=pl.BlockSpec((1, B, BN5), lambda i: (i, 0, 0)),
        out_shape=jax.ShapeDtypeStruct((NB5, B, BN5), _f32),
    )(c, w2, b2r, x5)


def kernel(input_seq, input_title, edge_index, g_node_feature, edge_type,
           edge_norm, emb_table, conv_w, conv_b, cf_w, cf_b,
           rgcn_basis0, rgcn_comp0, rgcn_bias0,
           rgcn_basis1, rgcn_comp1, rgcn_bias1,
           rgcn_basis2, rgcn_comp2, rgcn_bias2,
           cor_w1_0, cor_b1_0, cor_w2_0, cor_b2_0,
           cor_w1_1, cor_b1_1, cor_w2_1, cor_b2_1):
    LS = input_seq.shape[1]
    LT = input_title.shape[1]
    nseq = B * LS
    ntot = nseq + B * LT
    gpad = (-ntot) % (8 * NW)

    ids = jnp.concatenate([
        input_seq.reshape(-1), input_title.reshape(-1),
        jnp.zeros((gpad,), input_seq.dtype)]).astype(_i32)
    rows = _emb_gather(emb_table, ids)
    es = rows[:nseq].reshape(B, LS, EMB)
    et = rows[nseq:ntot].reshape(B, LT, EMB)

    wk = jnp.transpose(conv_w[:, 0], (1, 2, 0))
    cb2 = conv_b.reshape(1, NK)
    ac = _conv(es, wk, cb2)
    tcv = _conv(et, wk, cb2)

    src = edge_index[0].astype(_i32)
    dst = edge_index[1].astype(_i32)
    key = (edge_type.astype(_i32) * N + src).astype(_i32)
    norm = edge_norm[:, 0]

    pad_o = ((0, 0), (0, 0), (0, D - EMB))
    basis0 = jnp.pad(rgcn_basis0, pad_o)
    basis1 = jnp.pad(rgcn_basis1, ((0, 0), (0, D - EMB), (0, D - EMB)))
    basis2 = jnp.pad(rgcn_basis2, ((0, 0), (0, D - EMB), (0, D - EMB)))
    bias0 = jnp.pad(rgcn_bias0, (0, D - EMB)).reshape(1, D)
    bias1 = jnp.pad(rgcn_bias1, (0, D - EMB)).reshape(1, D)
    bias2 = jnp.pad(rgcn_bias2, (0, D - EMB)).reshape(1, D)

    hr0, hr1 = _rgcn_mm(g_node_feature, basis0, rgcn_comp0, bias0, first=True)
    p0 = _rgcn_scatter(hr0, hr1, key, dst, norm)
    hr0, hr1 = _rgcn_mm(p0, basis1, rgcn_comp1, bias0, first=False)
    p1 = _rgcn_scatter(hr0, hr1, key, dst, norm)
    hr0, hr1 = _rgcn_mm(p1, basis2, rgcn_comp2, bias1, first=False)
    p2 = _rgcn_scatter(hr0, hr1, key, dst, norm)

    x5 = _att(ac, tcv, g_node_feature, p2, bias2, cf_w,
              cf_b.reshape(1, 2 * EMB))

    for w1, b1, w2, b2, final in (
            (cor_w1_0, cor_b1_0, cor_w2_0, cor_b2_0, False),
            (cor_w1_1, cor_b1_1, cor_w2_1, cor_b2_1, True)):
        cblk = _cor_reduce(x5, w1.reshape(COR, NB5, BN5),
                           b1.reshape(COR // 200, 1, 200))
        c = cblk.transpose(1, 0, 2).reshape(B, COR)
        x5 = _cor_expand(c, w2, b2.reshape(NB5, 1, BN5), x5, final)

    return x5.transpose(1, 0, 2).reshape(B, N)


# speedup vs baseline: 3.8232x; 1.0026x over previous
import functools

import jax
import jax.numpy as jnp
from jax import lax
from jax.experimental import pallas as pl
from jax.experimental.pallas import tpu as pltpu
from jax.experimental.pallas import tpu_sc as plsc

N = 10000
EMB = 200
NK = 200
D = 256
DH = 128
E = 160000
B = 4
COR = 1000
NC = 2
NS = 16
NW = NC * NS
CH = 128
NCHUNK = E // CH
SUB_ROWS = 624
ZCH = 104

_f32 = jnp.float32
_i32 = jnp.int32


def _rgcn_scatter(hr0, hr1, key, dst, norm):
    mesh = plsc.VectorSubcoreMesh(core_axis_name="c", subcore_axis_name="s")

    @functools.partial(
        pl.kernel,
        out_type=jax.ShapeDtypeStruct((N, D), _f32),
        mesh=mesh,
        scratch_types=[
            pltpu.VMEM_SHARED((N, DH), _f32),
            pltpu.VMEM((CH,), _i32),
            pltpu.VMEM((CH,), _i32),
            pltpu.VMEM((CH,), _f32),
            pltpu.VMEM((CH, DH), _f32),
            pltpu.SemaphoreType.DMA,
        ],
    )
    def k(hr0_h, hr1_h, key_h, dst_h, norm_h, out_h,
          shared, keyv, dstv, normv, buf, sem):
        c = lax.axis_index("c")
        s = lax.axis_index("s")

        zero = jnp.zeros((16,), _f32)

        def zrow(r, _):
            for k2 in range(DH // 16):
                buf[r, pl.ds(k2 * 16, 16)] = zero
            return 0

        lax.fori_loop(0, ZCH, zrow, 0)
        row0 = s * SUB_ROWS
        for j in range(SUB_ROWS // ZCH):
            pltpu.sync_copy(buf.at[pl.ds(0, ZCH)],
                            shared.at[pl.ds(row0 + j * ZCH, ZCH)])

        @pl.when(s == NS - 1)
        def _():
            pltpu.sync_copy(buf.at[pl.ds(0, 16)],
                            shared.at[pl.ds(NS * SUB_ROWS, 16)])

        plsc.subcore_barrier()

        n_extra = NCHUNK - (NCHUNK // NS) * NS
        trips = jnp.where(s < n_extra, NCHUNK // NS + 1, NCHUNK // NS)

        def step(t, _):
            off = (s + t * NS) * CH
            pltpu.sync_copy(key_h.at[pl.ds(off, CH)], keyv)
            pltpu.sync_copy(dst_h.at[pl.ds(off, CH)], dstv)
            pltpu.sync_copy(norm_h.at[pl.ds(off, CH)], normv)

            @pl.when(c == 0)
            def _():
                pltpu.async_copy(hr0_h.at[keyv], buf, sem).wait()

            @pl.when(c == 1)
            def _():
                pltpu.async_copy(hr1_h.at[keyv], buf, sem).wait()

            def sgroup(g, _):
                nvv = normv[pl.ds(g * 16, 16)]
                dnums = lax.GatherDimensionNumbers(
                    offset_dims=(), collapsed_slice_dims=(0,),
                    start_index_map=(0,))
                for lane in range(16):
                    nv = lax.gather(
                        nvv, jnp.full((16, 1), lane, _i32), dnums,
                        slice_sizes=(1,),
                        mode=lax.GatherScatterMode.PROMISE_IN_BOUNDS)
                    r = g * 16 + lane
                    for k2 in range(DH // 16):
                        sl = pl.ds(k2 * 16, 16)
                        buf[r, sl] = buf[r, sl] * nv
                return 0

            lax.fori_loop(0, CH // 16, sgroup, 0)
            pltpu.sync_copy(buf, shared.at[dstv], add=True)
            return 0

        lax.fori_loop(0, trips, step, 0)
        plsc.subcore_barrier()

        col0 = c * DH
        for j in range(SUB_ROWS // ZCH):
            pltpu.sync_copy(
                shared.at[pl.ds(row0 + j * ZCH, ZCH)],
                out_h.at[pl.ds(row0 + j * ZCH, ZCH), pl.ds(col0, DH)])

        @pl.when(s == NS - 1)
        def _():
            pltpu.sync_copy(
                shared.at[pl.ds(NS * SUB_ROWS, 16)],
                out_h.at[pl.ds(NS * SUB_ROWS, 16), pl.ds(col0, DH)])

    return k(hr0, hr1, key, dst, norm)


def _conv(eseq, wk, bias):
    L = eseq.shape[1]
    LP = L - 2

    def body(e_ref, w_ref, b_ref, out_ref):
        for b in range(B):
            acc = jnp.dot(e_ref[b, 0:LP, :], w_ref[0],
                          preferred_element_type=_f32)
            acc += jnp.dot(e_ref[b, 1:LP + 1, :], w_ref[1],
                           preferred_element_type=_f32)
            acc += jnp.dot(e_ref[b, 2:LP + 2, :], w_ref[2],
                           preferred_element_type=_f32)
            acc += b_ref[...]
            out_ref[b] = jnp.maximum(acc, 0.0)

    return pl.pallas_call(
        body,
        out_shape=jax.ShapeDtypeStruct((B, LP, NK), _f32),
    )(eseq, wk, bias)


def _rgcn_mm(p, basis, comp, bias, first):
    R = 2000
    grid = (N // R,)

    def body(p_ref, basis_ref, comp_ref, bias_ref, o0_ref, o1_ref):
        if first:
            h = p_ref[...]
        else:
            h = jnp.maximum(p_ref[...] + bias_ref[...], 0.0)
        b0 = basis_ref[0]
        b1 = basis_ref[1]
        w0 = comp_ref[0, 0] * b0 + comp_ref[0, 1] * b1
        w1 = comp_ref[1, 0] * b0 + comp_ref[1, 1] * b1
        r0 = jnp.dot(h, w0, preferred_element_type=_f32)
        r1 = jnp.dot(h, w1, preferred_element_type=_f32)
        o0_ref[0] = r0[:, :DH]
        o0_ref[1] = r1[:, :DH]
        o1_ref[0] = r0[:, DH:]
        o1_ref[1] = r1[:, DH:]

    din = basis.shape[1]
    if first:
        p_spec = pl.BlockSpec((R, din), lambda i: (i, 0))
    else:
        p_spec = pl.BlockSpec((R, D), lambda i: (i, 0))
    half_spec = pl.BlockSpec((2, R, DH), lambda i: (0, i, 0))
    hr0, hr1 = pl.pallas_call(
        body,
        grid=grid,
        in_specs=[
            p_spec,
            pl.BlockSpec((2, din, D), lambda i: (0, 0, 0)),
            pl.BlockSpec((2, 2), lambda i: (0, 0)),
            pl.BlockSpec((1, D), lambda i: (0, 0)),
        ],
        out_specs=[half_spec, half_spec],
        out_shape=[jax.ShapeDtypeStruct((2, N, DH), _f32),
                   jax.ShapeDtypeStruct((2, N, DH), _f32)],
    )(p, basis, comp, bias)
    return hr0.reshape(2 * N, DH), hr1.reshape(2 * N, DH)


def _att(ac, tcv, g, p2, bias2, cf_w, cf_b):
    BN = 2000
    grid = (N // BN,)
    LA = ac.shape[1]
    LT = tcv.shape[1]

    def body(ac_ref, tc_ref, g_ref, p2_ref, b2_ref, cw_ref, cb_ref, out_ref):
        g_blk = g_ref[...]
        h3 = p2_ref[...] + b2_ref[...]
        lf = jnp.concatenate([h3[:, :EMB], g_blk], axis=1)
        cw = cw_ref[...]
        cb = cb_ref[...]
        for b in range(B):
            sa = lax.dot_general(ac_ref[b], g_blk,
                                 (((1,), (1,)), ((), ())),
                                 preferred_element_type=_f32)
            sa = jnp.exp(sa - jnp.max(sa, axis=0, keepdims=True))
            att_a = sa / jnp.sum(sa, axis=0, keepdims=True)
            ca = lax.dot_general(ac_ref[b], att_a,
                                 (((0,), (0,)), ((), ())),
                                 preferred_element_type=_f32)
            st = lax.dot_general(tc_ref[b], g_blk,
                                 (((1,), (1,)), ((), ())),
                                 preferred_element_type=_f32)
            st = jnp.exp(st - jnp.max(st, axis=0, keepdims=True))
            att_t = st / jnp.sum(st, axis=0, keepdims=True)
            ct = lax.dot_general(tc_ref[b], att_t,
                                 (((0,), (0,)), ((), ())),
                                 preferred_element_type=_f32)
            cc = jnp.concatenate([ca, ct], axis=0)
            xf = lax.dot_general(cc, cw,
                                 (((0,), (1,)), ((), ())),
                                 preferred_element_type=_f32)
            xf = jnp.tanh(xf + cb)
            out_ref[0, b, :] = jnp.sum(xf * lf, axis=1)

    return pl.pallas_call(
        body,
        grid=grid,
        in_specs=[
            pl.BlockSpec((B, LA, NK), lambda i: (0, 0, 0)),
            pl.BlockSpec((B, LT, NK), lambda i: (0, 0, 0)),
            pl.BlockSpec((BN, EMB), lambda i: (i, 0)),
            pl.BlockSpec((BN, D), lambda i: (i, 0)),
            pl.BlockSpec((1, D), lambda i: (0, 0)),
            pl.BlockSpec((2 * EMB, 2 * NK), lambda i: (0, 0)),
            pl.BlockSpec((1, 2 * EMB), lambda i: (0, 0)),
        ],
        out_specs=pl.BlockSpec((1, B, BN), lambda i: (i, 0, 0)),
        out_shape=jax.ShapeDtypeStruct((N // BN, B, BN), _f32),
    )(ac, tcv, g, p2, bias2, cf_w, cf_b)


NB5 = 5
BN5 = N // NB5


def _cor_reduce(x5, w1r, b1):
    CB = 200
    grid = (COR // CB,)

    def body(x_ref, w_ref, b_ref, out_ref):
        acc = jnp.zeros((B, CB), _f32)
        for k in range(NB5):
            o = jax.nn.sigmoid(x_ref[k])
            acc += lax.dot_general(o, w_ref[:, k, :],
                                   (((1,), (1,)), ((), ())),
                                   preferred_element_type=_f32)
        a = acc + b_ref[0, 0][None, :]
        out_ref[0] = jnp.where(a > 0, a, jnp.exp(a) - 1.0)

    return pl.pallas_call(
        body,
        grid=grid,
        in_specs=[
            pl.BlockSpec((NB5, B, BN5), lambda i: (0, 0, 0)),
            pl.BlockSpec((CB, NB5, BN5), lambda i: (i, 0, 0)),
            pl.BlockSpec((1, 1, CB), lambda i: (i, 0, 0)),
        ],
        out_specs=pl.BlockSpec((1, B, CB), lambda i: (i, 0, 0)),
        out_shape=jax.ShapeDtypeStruct((COR // CB, B, CB), _f32),
    )(x5, w1r, b1)


def _cor_expand(c, w2, b2r, x5, final):
    grid = (NB5,)

    def body(c_ref, w_ref, b_ref, x_ref, out_ref):
        r = lax.dot_general(c_ref[...], w_ref[...], (((1,), (1,)), ((), ())),
                            preferred_element_type=_f32)
        r = r + b_ref[0] + x_ref[0]
        if final:
            r = jax.nn.sigmoid(r)
        out_ref[0] = r

    return pl.pallas_call(
        body,
        grid=grid,
        in_specs=[
            pl.BlockSpec((B, COR), lambda i: (0, 0)),
            pl.BlockSpec((BN5, COR), lambda i: (i, 0)),
            pl.BlockSpec((1, 1, BN5), lambda i: (i, 0, 0)),
            pl.BlockSpec((1, B, BN5), lambda i: (i, 0, 0)),
        ],
        out_specs=pl.BlockSpec((1, B, BN5), lambda i: (i, 0, 0)),
        out_shape=jax.ShapeDtypeStruct((NB5, B, BN5), _f32),
    )(c, w2, b2r, x5)


def kernel(input_seq, input_title, edge_index, g_node_feature, edge_type,
           edge_norm, emb_table, conv_w, conv_b, cf_w, cf_b,
           rgcn_basis0, rgcn_comp0, rgcn_bias0,
           rgcn_basis1, rgcn_comp1, rgcn_bias1,
           rgcn_basis2, rgcn_comp2, rgcn_bias2,
           cor_w1_0, cor_b1_0, cor_w2_0, cor_b2_0,
           cor_w1_1, cor_b1_1, cor_w2_1, cor_b2_1):
    es = emb_table[input_seq]
    et = emb_table[input_title]

    wk = jnp.transpose(conv_w[:, 0], (1, 2, 0))
    cb2 = conv_b.reshape(1, NK)
    ac = _conv(es, wk, cb2)
    tcv = _conv(et, wk, cb2)

    src = edge_index[0].astype(_i32)
    dst = edge_index[1].astype(_i32)
    key = (edge_type.astype(_i32) * N + src).astype(_i32)
    norm = edge_norm[:, 0]

    pad_o = ((0, 0), (0, 0), (0, D - EMB))
    basis0 = jnp.pad(rgcn_basis0, pad_o)
    basis1 = jnp.pad(rgcn_basis1, ((0, 0), (0, D - EMB), (0, D - EMB)))
    basis2 = jnp.pad(rgcn_basis2, ((0, 0), (0, D - EMB), (0, D - EMB)))
    bias0 = jnp.pad(rgcn_bias0, (0, D - EMB)).reshape(1, D)
    bias1 = jnp.pad(rgcn_bias1, (0, D - EMB)).reshape(1, D)
    bias2 = jnp.pad(rgcn_bias2, (0, D - EMB)).reshape(1, D)

    hr0, hr1 = _rgcn_mm(g_node_feature, basis0, rgcn_comp0, bias0, first=True)
    p0 = _rgcn_scatter(hr0, hr1, key, dst, norm)
    hr0, hr1 = _rgcn_mm(p0, basis1, rgcn_comp1, bias0, first=False)
    p1 = _rgcn_scatter(hr0, hr1, key, dst, norm)
    hr0, hr1 = _rgcn_mm(p1, basis2, rgcn_comp2, bias1, first=False)
    p2 = _rgcn_scatter(hr0, hr1, key, dst, norm)

    x5 = _att(ac, tcv, g_node_feature, p2, bias2, cf_w,
              cf_b.reshape(1, 2 * EMB))

    for w1, b1, w2, b2, final in (
            (cor_w1_0, cor_b1_0, cor_w2_0, cor_b2_0, False),
            (cor_w1_1, cor_b1_1, cor_w2_1, cor_b2_1, True)):
        cblk = _cor_reduce(x5, w1.reshape(COR, NB5, BN5),
                           b1.reshape(COR // 200, 1, 200))
        c = cblk.transpose(1, 0, 2).reshape(B, COR)
        x5 = _cor_expand(c, w2, b2.reshape(NB5, 1, BN5), x5, final)

    return x5.transpose(1, 0, 2).reshape(B, N)


# speedup vs baseline: 5.0492x; 1.3207x over previous
import functools

import jax
import jax.numpy as jnp
from jax import lax
from jax.experimental import pallas as pl
from jax.experimental.pallas import tpu as pltpu
from jax.experimental.pallas import tpu_sc as plsc

N = 10000
EMB = 200
NK = 200
D = 256
DH = 128
E = 160000
B = 4
COR = 1000
NC = 2
NS = 16
NW = NC * NS
CH = 64
ECH = 2560
NCH = ECH // NS
NQ = NCH // 4
SUB_ROWS = 624
ZCH = 48
WCH = 104

_f32 = jnp.float32
_i32 = jnp.int32


def _emb_gather(t0, t1, ids):
    G = ids.shape[0]
    per = G // NW
    mesh = plsc.VectorSubcoreMesh(core_axis_name="c", subcore_axis_name="s")

    @functools.partial(
        pl.kernel,
        out_type=jax.ShapeDtypeStruct((G, 2 * DH), _f32),
        mesh=mesh,
        scratch_types=[
            pltpu.VMEM((per,), _i32),
            pltpu.VMEM((per, DH), _f32),
            pltpu.SemaphoreType.DMA,
        ],
    )
    def k(t0_h, t1_h, ids_h, out_h, idxv, rowsv, sem):
        c = lax.axis_index("c")
        s = lax.axis_index("s")
        w = c * NS + s
        base = w * per
        pltpu.sync_copy(ids_h.at[pl.ds(base, per)], idxv)
        pltpu.async_copy(t0_h.at[idxv], rowsv, sem).wait()
        pltpu.sync_copy(rowsv, out_h.at[pl.ds(base, per), pl.ds(0, DH)])
        pltpu.async_copy(t1_h.at[idxv], rowsv, sem).wait()
        pltpu.sync_copy(rowsv, out_h.at[pl.ds(base, per), pl.ds(DH, DH)])

    return k(t0, t1, ids)


def _rgcn_scatter(hr0, hr1, packed, norm2):
    mesh = plsc.VectorSubcoreMesh(core_axis_name="c", subcore_axis_name="s")

    @functools.partial(
        pl.kernel,
        out_type=jax.ShapeDtypeStruct((N, D), _f32),
        mesh=mesh,
        scratch_types=[
            pltpu.VMEM_SHARED((N, DH), _f32),
            pltpu.VMEM((CH, DH), _f32),
            pltpu.VMEM((CH, DH), _f32),
            pltpu.VMEM((CH, DH), _f32),
            pltpu.VMEM((CH, DH), _f32),
        ] + [pltpu.VMEM((2, CH), _i32)] * 8
          + [pltpu.VMEM((CH,), _f32)] * 8
          + [pltpu.SemaphoreType.DMA] * 4
          + [pltpu.SemaphoreType.DMA] * 4
          + [pltpu.SemaphoreType.DMA] * 8,
    )
    def k(hr0_h, hr1_h, packed_h, norm_h, out_h,
          shared, b0, b1, b2, b3,
          i0, i1, i2, i3, i4, i5, i6, i7,
          n0, n1, n2, n3, n4, n5, n6, n7,
          g0, g1, g2, g3, s0, s1, s2, s3,
          q0, q1, q2, q3, q4, q5, q6, q7):
        c = lax.axis_index("c")
        s = lax.axis_index("s")
        bufs = (b0, b1, b2, b3)
        idxs = (i0, i1, i2, i3, i4, i5, i6, i7)
        norms = (n0, n1, n2, n3, n4, n5, n6, n7)
        gsems = (g0, g1, g2, g3)
        ssems = (s0, s1, s2, s3)
        qsems = (q0, q1, q2, q3, q4, q5, q6, q7)
        base = s * NCH

        zero = jnp.zeros((16,), _f32)

        def zrow(r, _):
            for k2 in range(DH // 16):
                b0[r, pl.ds(k2 * 16, 16)] = zero
            return 0

        lax.fori_loop(0, ZCH, zrow, 0)
        row0 = s * SUB_ROWS
        for j in range(SUB_ROWS // ZCH):
            pltpu.sync_copy(b0.at[pl.ds(0, ZCH)],
                            shared.at[pl.ds(row0 + j * ZCH, ZCH)])

        @pl.when(s == NS - 1)
        def _():
            pltpu.sync_copy(b0.at[pl.ds(0, 16)],
                            shared.at[pl.ds(NS * SUB_ROWS, 16)])

        plsc.subcore_barrier()

        def fire_i(sc_idx, ii):
            pltpu.async_copy(packed_h.at[base + sc_idx], idxs[ii], qsems[ii])
            pltpu.async_copy(norm_h.at[base + sc_idx], norms[ii], qsems[ii])

        def wait_i(ii):
            pltpu.make_async_copy(packed_h.at[0], idxs[ii], qsems[ii]).wait()
            pltpu.make_async_copy(norm_h.at[0], norms[ii], qsems[ii]).wait()

        def fire_g(ii, bi):
            @pl.when(c == 0)
            def _():
                pltpu.async_copy(hr0_h.at[idxs[ii].at[0]], bufs[bi],
                                 gsems[bi])

            @pl.when(c == 1)
            def _():
                pltpu.async_copy(hr1_h.at[idxs[ii].at[0]], bufs[bi],
                                 gsems[bi])

        def wait_g(bi):
            pltpu.make_async_copy(hr0_h.at[pl.ds(0, CH)], bufs[bi],
                                  gsems[bi]).wait()

        def fire_s(ii, bi):
            pltpu.async_copy(bufs[bi], shared.at[idxs[ii].at[1]],
                             ssems[bi], add=True)

        def wait_s(bi):
            pltpu.make_async_copy(hr0_h.at[pl.ds(0, CH)], bufs[bi],
                                  ssems[bi]).wait()

        dnums = lax.GatherDimensionNumbers(
            offset_dims=(), collapsed_slice_dims=(0,), start_index_map=(0,))

        def scale(ii, bi):
            buf = bufs[bi]
            nrm = norms[ii]

            def sgroup(g, _):
                nvv = nrm[pl.ds(g * 16, 16)]
                for lane in range(16):
                    nv = lax.gather(
                        nvv, jnp.full((16, 1), lane, _i32), dnums,
                        slice_sizes=(1,),
                        mode=lax.GatherScatterMode.PROMISE_IN_BOUNDS)
                    r = g * 16 + lane
                    for k2 in range(DH // 16):
                        sl = pl.ds(k2 * 16, 16)
                        buf[r, sl] = buf[r, sl] * nv
                return 0

            lax.fori_loop(0, CH // 16, sgroup, 0)

        fire_i(0, 0)
        fire_i(1, 1)
        fire_i(2, 2)
        wait_i(0)
        fire_g(0, 0)
        wait_i(1)
        fire_g(1, 1)

        def ring(q, _):
            for o in range(8):
                sc_idx = q * 8 + o
                pre = sc_idx + 3
                nxt = sc_idx + 2
                ii = o
                bi = o % 4
                ip = (o + 3) % 8
                iw = (o + 2) % 8
                bn = (o + 2) % 4

                @pl.when(pre < NCH)
                def _():
                    fire_i(pre, ip)

                @pl.when(nxt < NCH)
                def _():
                    @pl.when(nxt >= 4)
                    def _():
                        wait_s(bn)

                    wait_i(iw)
                    fire_g(iw, bn)

                wait_g(bi)
                scale(ii, bi)
                fire_s(ii, bi)
            return 0

        lax.fori_loop(0, NCH // 8, ring, 0)
        for bi2 in range(4):
            wait_s(bi2)
        plsc.subcore_barrier()

        col0 = c * DH
        for j in range(SUB_ROWS // ZCH):
            pltpu.sync_copy(
                shared.at[pl.ds(row0 + j * ZCH, ZCH)],
                out_h.at[pl.ds(row0 + j * ZCH, ZCH), pl.ds(col0, DH)])

        @pl.when(s == NS - 1)
        def _():
            pltpu.sync_copy(
                shared.at[pl.ds(NS * SUB_ROWS, 16)],
                out_h.at[pl.ds(NS * SUB_ROWS, 16), pl.ds(col0, DH)])

    return k(hr0, hr1, packed, norm2)


def _conv(eseq, wk, bias):
    L = eseq.shape[1]
    LP = L - 2

    def body(e_ref, w_ref, b_ref, out_ref):
        for b in range(B):
            acc = jnp.dot(e_ref[b, 0:LP, :], w_ref[0], precision=lax.Precision.HIGHEST,
                          preferred_element_type=_f32)
            acc += jnp.dot(e_ref[b, 1:LP + 1, :], w_ref[1], precision=lax.Precision.HIGHEST,
                           preferred_element_type=_f32)
            acc += jnp.dot(e_ref[b, 2:LP + 2, :], w_ref[2], precision=lax.Precision.HIGHEST,
                           preferred_element_type=_f32)
            acc += b_ref[...]
            out_ref[b] = jnp.maximum(acc, 0.0)

    return pl.pallas_call(
        body,
        out_shape=jax.ShapeDtypeStruct((B, LP, NK), _f32),
    )(eseq, wk, bias)


def _rgcn_mm(p, basis, comp, bias, first):
    R = 2000
    grid = (N // R,)

    def body(p_ref, basis_ref, comp_ref, bias_ref, o0_ref, o1_ref):
        if first:
            h = p_ref[...]
        else:
            h = jnp.maximum(p_ref[...] + bias_ref[...], 0.0)
        b0 = basis_ref[0]
        b1 = basis_ref[1]
        w0 = comp_ref[0, 0] * b0 + comp_ref[0, 1] * b1
        w1 = comp_ref[1, 0] * b0 + comp_ref[1, 1] * b1
        r0 = jnp.dot(h, w0, preferred_element_type=_f32,
                     precision=lax.Precision.HIGHEST)
        r1 = jnp.dot(h, w1, preferred_element_type=_f32,
                     precision=lax.Precision.HIGHEST)
        o0_ref[0] = r0[:, :DH]
        o0_ref[1] = r1[:, :DH]
        o1_ref[0] = r0[:, DH:]
        o1_ref[1] = r1[:, DH:]

    din = basis.shape[1]
    if first:
        p_spec = pl.BlockSpec((R, din), lambda i: (i, 0))
    else:
        p_spec = pl.BlockSpec((R, D), lambda i: (i, 0))
    half_spec = pl.BlockSpec((2, R, DH), lambda i: (0, i, 0))
    hr0, hr1 = pl.pallas_call(
        body,
        grid=grid,
        in_specs=[
            p_spec,
            pl.BlockSpec((2, din, D), lambda i: (0, 0, 0)),
            pl.BlockSpec((2, 2), lambda i: (0, 0)),
            pl.BlockSpec((1, D), lambda i: (0, 0)),
        ],
        out_specs=[half_spec, half_spec],
        out_shape=[jax.ShapeDtypeStruct((2, N, DH), _f32),
                   jax.ShapeDtypeStruct((2, N, DH), _f32)],
    )(p, basis, comp, bias)
    return hr0.reshape(2 * N, DH), hr1.reshape(2 * N, DH)


def _att1(ac, tcv, g, cf_w, cf_b):
    BN = 2000
    grid = (N // BN,)
    LA = ac.shape[1]
    LT = tcv.shape[1]

    def body(ac_ref, tc_ref, g_ref, cw_ref, cb_ref, out_ref):
        g_blk = g_ref[...]
        cw = cw_ref[...]
        cb = cb_ref[...]
        for b in range(B):
            sa = lax.dot_general(ac_ref[b], g_blk,
                                 (((1,), (1,)), ((), ())),
                                 preferred_element_type=_f32,
                                 precision=lax.Precision.HIGHEST)
            sa = jnp.exp(sa - jnp.max(sa, axis=0, keepdims=True))
            att_a = sa / jnp.sum(sa, axis=0, keepdims=True)
            ca = lax.dot_general(ac_ref[b], att_a,
                                 (((0,), (0,)), ((), ())),
                                 preferred_element_type=_f32,
                                 precision=lax.Precision.HIGHEST)
            st = lax.dot_general(tc_ref[b], g_blk,
                                 (((1,), (1,)), ((), ())),
                                 preferred_element_type=_f32,
                                 precision=lax.Precision.HIGHEST)
            st = jnp.exp(st - jnp.max(st, axis=0, keepdims=True))
            att_t = st / jnp.sum(st, axis=0, keepdims=True)
            ct = lax.dot_general(tc_ref[b], att_t,
                                 (((0,), (0,)), ((), ())),
                                 preferred_element_type=_f32,
                                 precision=lax.Precision.HIGHEST)
            cc = jnp.concatenate([ca, ct], axis=0)
            xf = lax.dot_general(cc, cw,
                                 (((0,), (1,)), ((), ())),
                                 preferred_element_type=_f32,
                                 precision=lax.Precision.HIGHEST)
            out_ref[b] = jnp.tanh(xf + cb)

    return pl.pallas_call(
        body,
        grid=grid,
        in_specs=[
            pl.BlockSpec((B, LA, NK), lambda i: (0, 0, 0)),
            pl.BlockSpec((B, LT, NK), lambda i: (0, 0, 0)),
            pl.BlockSpec((BN, EMB), lambda i: (i, 0)),
            pl.BlockSpec((2 * EMB, 2 * NK), lambda i: (0, 0)),
            pl.BlockSpec((1, 2 * EMB), lambda i: (0, 0)),
        ],
        out_specs=pl.BlockSpec((B, BN, 2 * EMB), lambda i: (0, i, 0)),
        out_shape=jax.ShapeDtypeStruct((B, N, 2 * EMB), _f32),
    )(ac, tcv, g, cf_w, cf_b)


def _att2(xf, g, p2, bias2):
    BN = 2000
    grid = (N // BN,)

    def body(xf_ref, g_ref, p2_ref, b2_ref, out_ref):
        h3 = p2_ref[...] + b2_ref[...]
        lf = jnp.concatenate([h3[:, :EMB], g_ref[...]], axis=1)
        for b in range(B):
            out_ref[0, b, :] = jnp.sum(xf_ref[b] * lf, axis=1)

    return pl.pallas_call(
        body,
        grid=grid,
        in_specs=[
            pl.BlockSpec((B, BN, 2 * EMB), lambda i: (0, i, 0)),
            pl.BlockSpec((BN, EMB), lambda i: (i, 0)),
            pl.BlockSpec((BN, D), lambda i: (i, 0)),
            pl.BlockSpec((1, D), lambda i: (0, 0)),
        ],
        out_specs=pl.BlockSpec((1, B, BN), lambda i: (i, 0, 0)),
        out_shape=jax.ShapeDtypeStruct((N // BN, B, BN), _f32),
    )(xf, g, p2, bias2)


NB5 = 5
BN5 = N // NB5


def _cor_reduce(x5, w1, b1):
    CB = 200
    grid = (COR // CB,)

    def body(x_ref, w_ref, b_ref, out_ref):
        o = jax.nn.sigmoid(
            jnp.concatenate([x_ref[k] for k in range(NB5)], axis=1))
        acc = lax.dot_general(o, w_ref[...], (((1,), (1,)), ((), ())),
                              preferred_element_type=_f32, precision=lax.Precision.HIGHEST)
        a = acc + b_ref[0]
        out_ref[0] = jnp.where(a > 0, a, jnp.exp(a) - 1.0)

    return pl.pallas_call(
        body,
        grid=grid,
        in_specs=[
            pl.BlockSpec((NB5, B, BN5), lambda i: (0, 0, 0)),
            pl.BlockSpec((CB, N), lambda i: (i, 0)),
            pl.BlockSpec((1, 1, CB), lambda i: (i, 0, 0)),
        ],
        out_specs=pl.BlockSpec((1, B, CB), lambda i: (i, 0, 0)),
        out_shape=jax.ShapeDtypeStruct((COR // CB, B, CB), _f32),
    )(x5, w1, b1)


def _cor_expand(c, w2, b2r, x5, final):
    grid = (NB5,)

    def body(c_ref, w_ref, b_ref, x_ref, out_ref):
        r = lax.dot_general(c_ref[...], w_ref[...], (((1,), (1,)), ((), ())),
                            preferred_element_type=_f32, precision=lax.Precision.HIGHEST)
        r = r + b_ref[0] + x_ref[0]
        if final:
            r = jax.nn.sigmoid(r)
        out_ref[0] = r

    return pl.pallas_call(
        body,
        grid=grid,
        in_specs=[
            pl.BlockSpec((B, COR), lambda i: (0, 0)),
            pl.BlockSpec((BN5, COR), lambda i: (i, 0)),
            pl.BlockSpec((1, 1, BN5), lambda i: (i, 0, 0)),
            pl.BlockSpec((1, B, BN5), lambda i: (i, 0, 0)),
        ],
        out_specs=pl.BlockSpec((1, B, BN5), lambda i: (i, 0, 0)),
        out_shape=jax.ShapeDtypeStruct((NB5, B, BN5), _f32),
    )(c, w2, b2r, x5)


def kernel(input_seq, input_title, edge_index, g_node_feature, edge_type,
           edge_norm, emb_table, conv_w, conv_b, cf_w, cf_b,
           rgcn_basis0, rgcn_comp0, rgcn_bias0,
           rgcn_basis1, rgcn_comp1, rgcn_bias1,
           rgcn_basis2, rgcn_comp2, rgcn_bias2,
           cor_w1_0, cor_b1_0, cor_w2_0, cor_b2_0,
           cor_w1_1, cor_b1_1, cor_w2_1, cor_b2_1):
    LS = input_seq.shape[1]
    LT2 = input_title.shape[1]
    nseq = B * LS
    ntot = nseq + B * LT2
    gpad = (-ntot) % (8 * NW)
    ids = jnp.concatenate([
        input_seq.reshape(-1), input_title.reshape(-1),
        jnp.zeros((gpad,), input_seq.dtype)]).astype(_i32)
    t0 = emb_table[:, :DH]
    t1 = jnp.pad(emb_table[:, DH:], ((0, 0), (0, 2 * DH - EMB)))
    rows = _emb_gather(t0, t1, ids)
    es = rows[:nseq, :EMB].reshape(B, LS, EMB)
    et = rows[nseq:ntot, :EMB].reshape(B, LT2, EMB)

    wk = jnp.transpose(conv_w[:, 0], (1, 2, 0))
    cb2 = conv_b.reshape(1, NK)
    ac = _conv(es, wk, cb2)
    tcv = _conv(et, wk, cb2)

    src = edge_index[0].astype(_i32)
    dst = edge_index[1].astype(_i32)
    key = (edge_type.astype(_i32) * N + src).astype(_i32)
    norm = edge_norm[:, 0]
    epad = ECH * CH - E
    key2 = jnp.pad(key, (0, epad)).reshape(ECH, CH)
    dst2 = jnp.pad(dst, (0, epad)).reshape(ECH, CH)
    norm2 = jnp.pad(norm, (0, epad)).reshape(ECH, CH)
    packed = jnp.stack([key2, dst2], axis=1)

    pad_o = ((0, 0), (0, 0), (0, D - EMB))
    basis0 = jnp.pad(rgcn_basis0, pad_o)
    basis1 = jnp.pad(rgcn_basis1, ((0, 0), (0, D - EMB), (0, D - EMB)))
    basis2 = jnp.pad(rgcn_basis2, ((0, 0), (0, D - EMB), (0, D - EMB)))
    bias0 = jnp.pad(rgcn_bias0, (0, D - EMB)).reshape(1, D)
    bias1 = jnp.pad(rgcn_bias1, (0, D - EMB)).reshape(1, D)
    bias2 = jnp.pad(rgcn_bias2, (0, D - EMB)).reshape(1, D)

    xf = _att1(ac, tcv, g_node_feature, cf_w, cf_b.reshape(1, 2 * EMB))

    hr0, hr1 = _rgcn_mm(g_node_feature, basis0, rgcn_comp0, bias0, first=True)
    p0 = _rgcn_scatter(hr0, hr1, packed, norm2)
    hr0, hr1 = _rgcn_mm(p0, basis1, rgcn_comp1, bias0, first=False)
    p1 = _rgcn_scatter(hr0, hr1, packed, norm2)
    hr0, hr1 = _rgcn_mm(p1, basis2, rgcn_comp2, bias1, first=False)
    p2 = _rgcn_scatter(hr0, hr1, packed, norm2)

    x5 = _att2(xf, g_node_feature, p2, bias2)

    for w1, b1, w2, b2, final in (
            (cor_w1_0, cor_b1_0, cor_w2_0, cor_b2_0, False),
            (cor_w1_1, cor_b1_1, cor_w2_1, cor_b2_1, True)):
        cblk = _cor_reduce(x5, w1, b1.reshape(COR // 200, 1, 200))
        c = cblk.transpose(1, 0, 2).reshape(B, COR)
        x5 = _cor_expand(c, w2, b2.reshape(NB5, 1, BN5), x5, final)

    return x5.transpose(1, 0, 2).reshape(B, N)
